# baseline (speedup 1.0000x reference)
import functools

import jax
import jax.numpy as jnp
from jax import lax
from jax.experimental import pallas as pl
from jax.experimental.pallas import tpu as pltpu
from jax.experimental.pallas import tpu_sc as plsc

N_NODES = 10000
N_TAB = 10240
D_IN = 128
D_AUG = 144
N_EDGES = 320000
NUM_WORKERS = 32
CHUNK = 128
NUM_CHUNKS = 80
EDGES_PER_WORKER = NUM_CHUNKS * CHUNK
E_PAD = NUM_WORKERS * EDGES_PER_WORKER
ROWS_PER_SUBCORE = N_NODES // 16


def _sc_segment_sum(xa, src3, dst3, zblk):
    mesh = plsc.VectorSubcoreMesh(core_axis_name="c", subcore_axis_name="s")

    @functools.partial(
        pl.kernel,
        out_type=jax.ShapeDtypeStruct((2, N_NODES, D_AUG), jnp.float32),
        mesh=mesh,
        scratch_types=[
            pltpu.VMEM((NUM_CHUNKS, CHUNK), jnp.int32),
            pltpu.VMEM((NUM_CHUNKS, CHUNK), jnp.int32),
            pltpu.VMEM((CHUNK, D_AUG), jnp.float32),
            pltpu.VMEM_SHARED((N_NODES, D_AUG), jnp.float32),
            pltpu.SemaphoreType.DMA,
        ],
        compiler_params=pltpu.CompilerParams(use_tc_tiling_on_sc=False),
    )
    def seg_sum(xa_hbm, src_hbm, dst_hbm, zblk_hbm, out_hbm,
                src_v, dst_v, rows_v, acc_sh, gsem):
        c = lax.axis_index("c")
        s = lax.axis_index("s")
        wid = s * 2 + c
        row0 = s * ROWS_PER_SUBCORE

        pltpu.sync_copy(zblk_hbm, acc_sh.at[pl.ds(row0, ROWS_PER_SUBCORE), :])
        pltpu.sync_copy(src_hbm.at[wid], src_v)
        pltpu.sync_copy(dst_hbm.at[wid], dst_v)
        plsc.subcore_barrier()

        def chunk_body(g, carry):
            pltpu.async_copy(xa_hbm.at[src_v.at[g]], rows_v, gsem).wait()
            pltpu.sync_copy(rows_v, acc_sh.at[dst_v.at[g]], add=True)
            return carry

        lax.fori_loop(0, NUM_CHUNKS, chunk_body, 0)
        plsc.subcore_barrier()

        pltpu.sync_copy(acc_sh.at[pl.ds(row0, ROWS_PER_SUBCORE), :],
                        out_hbm.at[c, pl.ds(row0, ROWS_PER_SUBCORE), :])

    return seg_sum(xa, src3, dst3, zblk)


def _tc_dense_body(x_ref, a_ref, wm_ref, bm_ref, wu_ref, bu_ref, o_ref):
    asum = a_ref[0] + a_ref[1]
    feat = asum[:, :D_IN]
    cnt = asum[:, D_IN:D_IN + 1]
    inv = 1.0 / jnp.maximum(cnt, 1.0)
    gate = cnt * inv
    w1 = wm_ref[:, :D_IN]
    w2 = wm_ref[:, D_IN:]
    dn = (((1,), (1,)), ((), ()))
    t1 = lax.dot_general(feat, w1, dn, preferred_element_type=jnp.float32)
    t2 = lax.dot_general(x_ref[...], w2, dn, preferred_element_type=jnp.float32)
    msgs = t1 * inv + gate * (t2 + bm_ref[...])
    out = lax.dot_general(msgs, wu_ref[...], dn, preferred_element_type=jnp.float32)
    o_ref[...] = out + bu_ref[...]


def _tc_dense(xb, acc, W_msg, b_msg, W_upd, b_upd):
    blk = 1000
    grid = N_NODES // blk
    return pl.pallas_call(
        _tc_dense_body,
        grid=(grid,),
        in_specs=[
            pl.BlockSpec((blk, D_IN), lambda i: (i, 0)),
            pl.BlockSpec((2, blk, D_AUG), lambda i: (0, i, 0)),
            pl.BlockSpec((D_IN, 2 * D_IN), lambda i: (0, 0)),
            pl.BlockSpec((1, D_IN), lambda i: (0, 0)),
            pl.BlockSpec((D_IN, D_IN), lambda i: (0, 0)),
            pl.BlockSpec((1, D_IN), lambda i: (0, 0)),
        ],
        out_specs=pl.BlockSpec((blk, D_IN), lambda i: (i, 0)),
        out_shape=jax.ShapeDtypeStruct((N_NODES, D_IN), jnp.float32),
    )(xb, acc, W_msg, b_msg, W_upd, b_upd)


@jax.jit
def kernel(x, edge_index, W_msg, b_msg, W_upd, b_upd):
    xb = x[0]
    src = edge_index[0].astype(jnp.int32)
    dst = edge_index[1].astype(jnp.int32)
    npad_e = E_PAD - N_EDGES
    src3 = jnp.concatenate(
        [src, jnp.full((npad_e,), N_TAB - 1, jnp.int32)]).reshape(
        NUM_WORKERS, NUM_CHUNKS, CHUNK)
    dst3 = jnp.concatenate([dst, jnp.zeros((npad_e,), jnp.int32)]).reshape(
        NUM_WORKERS, NUM_CHUNKS, CHUNK)

    xa = jnp.zeros((N_TAB, D_AUG), jnp.float32)
    xa = xa.at[:N_NODES, :D_IN].set(xb)
    xa = xa.at[:N_NODES, D_IN].set(1.0)
    zblk = jnp.zeros((ROWS_PER_SUBCORE, D_AUG), jnp.float32)

    acc = _sc_segment_sum(xa, src3, dst3, zblk)

    out = _tc_dense(xb, acc, W_msg,
                    b_msg.reshape(1, D_IN), W_upd, b_upd.reshape(1, D_IN))
    return out[None]


# speedup vs baseline: 1.0767x; 1.0767x over previous
import functools

import jax
import jax.numpy as jnp
from jax import lax
from jax.experimental import pallas as pl
from jax.experimental.pallas import tpu as pltpu
from jax.experimental.pallas import tpu_sc as plsc

N_NODES = 10000
N_TAB = 10240
D_IN = 128
D_AUG = 144
N_EDGES = 320000
NUM_WORKERS = 32
CHUNK = 128
NUM_CHUNKS = 80
EDGES_PER_WORKER = NUM_CHUNKS * CHUNK
E_PAD = NUM_WORKERS * EDGES_PER_WORKER
ROWS_PER_SUBCORE = N_NODES // 16


def _sc_segment_sum(xa, src3, dst3, zblk):
    mesh = plsc.VectorSubcoreMesh(core_axis_name="c", subcore_axis_name="s")

    @functools.partial(
        pl.kernel,
        out_type=jax.ShapeDtypeStruct((2, N_NODES, D_AUG), jnp.float32),
        mesh=mesh,
        scratch_types=[
            pltpu.VMEM((NUM_CHUNKS, CHUNK), jnp.int32),
            pltpu.VMEM((NUM_CHUNKS, CHUNK), jnp.int32),
            pltpu.VMEM((CHUNK, D_AUG), jnp.float32),
            pltpu.VMEM_SHARED((N_NODES, D_AUG), jnp.float32),
            pltpu.SemaphoreType.DMA,
        ],
        compiler_params=pltpu.CompilerParams(use_tc_tiling_on_sc=False),
    )
    def seg_sum(xa_hbm, src_hbm, dst_hbm, zblk_hbm, out_hbm,
                src_v, dst_v, rows_v, acc_sh, gsem):
        c = lax.axis_index("c")
        s = lax.axis_index("s")
        wid = s * 2 + c
        row0 = s * ROWS_PER_SUBCORE

        pltpu.sync_copy(zblk_hbm, acc_sh.at[pl.ds(row0, ROWS_PER_SUBCORE), :])
        pltpu.sync_copy(src_hbm.at[wid], src_v)
        pltpu.sync_copy(dst_hbm.at[wid], dst_v)
        plsc.subcore_barrier()

        def chunk_body(g, carry):
            pltpu.async_copy(xa_hbm.at[src_v.at[g]], rows_v, gsem).wait()
            pltpu.sync_copy(rows_v, acc_sh.at[dst_v.at[g]], add=True)
            return carry

        lax.fori_loop(0, NUM_CHUNKS, chunk_body, 0)
        plsc.subcore_barrier()

        pltpu.sync_copy(acc_sh.at[pl.ds(row0, ROWS_PER_SUBCORE), :],
                        out_hbm.at[c, pl.ds(row0, ROWS_PER_SUBCORE), :])

    return seg_sum(xa, src3, dst3, zblk)


def _tc_dense_body(x_ref, a_ref, wm_ref, bm_ref, wu_ref, bu_ref, o_ref):
    asum = a_ref[0] + a_ref[1]
    feat = asum[:, :D_IN]
    cnt = asum[:, D_IN:D_IN + 1]
    inv = 1.0 / jnp.maximum(cnt, 1.0)
    gate = cnt * inv
    w1 = wm_ref[:, :D_IN]
    w2 = wm_ref[:, D_IN:]
    dn = (((1,), (1,)), ((), ()))
    t1 = lax.dot_general(feat, w1, dn, preferred_element_type=jnp.float32)
    t2 = lax.dot_general(x_ref[...], w2, dn, preferred_element_type=jnp.float32)
    msgs = t1 * inv + gate * (t2 + bm_ref[...])
    out = lax.dot_general(msgs, wu_ref[...], dn, preferred_element_type=jnp.float32)
    o_ref[...] = out + bu_ref[...]


def _tc_dense(xb, acc, W_msg, b_msg, W_upd, b_upd):
    blk = 1000
    grid = N_NODES // blk
    return pl.pallas_call(
        _tc_dense_body,
        grid=(grid,),
        in_specs=[
            pl.BlockSpec((blk, D_IN), lambda i: (i, 0)),
            pl.BlockSpec((2, blk, D_AUG), lambda i: (0, i, 0)),
            pl.BlockSpec((D_IN, 2 * D_IN), lambda i: (0, 0)),
            pl.BlockSpec((1, D_IN), lambda i: (0, 0)),
            pl.BlockSpec((D_IN, D_IN), lambda i: (0, 0)),
            pl.BlockSpec((1, D_IN), lambda i: (0, 0)),
        ],
        out_specs=pl.BlockSpec((blk, D_IN), lambda i: (i, 0)),
        out_shape=jax.ShapeDtypeStruct((N_NODES, D_IN), jnp.float32),
    )(xb, acc, W_msg, b_msg, W_upd, b_upd)


@jax.jit
def kernel(x, edge_index, W_msg, b_msg, W_upd, b_upd):
    xb = x[0]
    src = edge_index[0].astype(jnp.int32)
    dst = edge_index[1].astype(jnp.int32)
    npad_e = E_PAD - N_EDGES
    src3 = jnp.concatenate(
        [src, jnp.full((npad_e,), N_TAB - 1, jnp.int32)]).reshape(
        NUM_WORKERS, NUM_CHUNKS, CHUNK)
    dst3 = jnp.concatenate(
        [dst, jnp.arange(npad_e, dtype=jnp.int32) % N_NODES]).reshape(
        NUM_WORKERS, NUM_CHUNKS, CHUNK)

    xa = jnp.zeros((N_TAB, D_AUG), jnp.float32)
    xa = xa.at[:N_NODES, :D_IN].set(xb)
    xa = xa.at[:N_NODES, D_IN].set(1.0)
    zblk = jnp.zeros((ROWS_PER_SUBCORE, D_AUG), jnp.float32)

    acc = _sc_segment_sum(xa, src3, dst3, zblk)

    out = _tc_dense(xb, acc, W_msg,
                    b_msg.reshape(1, D_IN), W_upd, b_upd.reshape(1, D_IN))
    return out[None]


# speedup vs baseline: 1.5371x; 1.4276x over previous
import functools

import jax
import jax.numpy as jnp
from jax import lax
from jax.experimental import pallas as pl
from jax.experimental.pallas import tpu as pltpu
from jax.experimental.pallas import tpu_sc as plsc

N_NODES = 10000
N_TAB = 10240
D_IN = 128
D_AUG = 144
N_EDGES = 320000
NUM_WORKERS = 32
CHUNK = 80
NUM_CHUNKS = 126
EDGES_PER_WORKER = NUM_CHUNKS * CHUNK
E_PAD = NUM_WORKERS * EDGES_PER_WORKER
ROWS_PER_SUBCORE = N_NODES // 16


def _sc_segment_sum(xa, src3, dst3, zblk):
    mesh = plsc.VectorSubcoreMesh(core_axis_name="c", subcore_axis_name="s")

    @functools.partial(
        pl.kernel,
        out_type=jax.ShapeDtypeStruct((2, N_NODES, D_AUG), jnp.float32),
        mesh=mesh,
        scratch_types=[
            pltpu.VMEM((NUM_CHUNKS, CHUNK), jnp.int32),
            pltpu.VMEM((NUM_CHUNKS, CHUNK), jnp.int32),
            pltpu.VMEM((CHUNK, D_AUG), jnp.float32),
            pltpu.VMEM_SHARED((N_NODES, D_AUG), jnp.float32),
            pltpu.SemaphoreType.DMA,
        ],
        compiler_params=pltpu.CompilerParams(use_tc_tiling_on_sc=False),
    )
    def seg_sum(xa_hbm, src_hbm, dst_hbm, zblk_hbm, out_hbm,
                src_v, dst_v, rows_v, acc_sh, gsem):
        c = lax.axis_index("c")
        s = lax.axis_index("s")
        wid = s * 2 + c
        row0 = s * ROWS_PER_SUBCORE

        pltpu.sync_copy(zblk_hbm, acc_sh.at[pl.ds(row0, ROWS_PER_SUBCORE), :])
        pltpu.sync_copy(src_hbm.at[wid], src_v)
        pltpu.sync_copy(dst_hbm.at[wid], dst_v)
        plsc.subcore_barrier()

        def chunk_body(g, carry):
            pltpu.async_copy(xa_hbm.at[src_v.at[g]], rows_v, gsem).wait()
            pltpu.sync_copy(rows_v, acc_sh.at[dst_v.at[g]], add=True)
            return carry

        lax.fori_loop(0, NUM_CHUNKS, chunk_body, 0)
        plsc.subcore_barrier()

        pltpu.sync_copy(acc_sh.at[pl.ds(row0, ROWS_PER_SUBCORE), :],
                        out_hbm.at[c, pl.ds(row0, ROWS_PER_SUBCORE), :])

    return seg_sum(xa, src3, dst3, zblk)


def _tc_dense_body(x_ref, a_ref, wm_ref, bm_ref, wu_ref, bu_ref, o_ref):
    asum = a_ref[0] + a_ref[1]
    feat = asum[:, :D_IN]
    cnt = asum[:, D_IN:D_IN + 1]
    inv = 1.0 / jnp.maximum(cnt, 1.0)
    gate = cnt * inv
    w1 = wm_ref[:, :D_IN]
    w2 = wm_ref[:, D_IN:]
    dn = (((1,), (1,)), ((), ()))
    t1 = lax.dot_general(feat, w1, dn, preferred_element_type=jnp.float32)
    t2 = lax.dot_general(x_ref[...], w2, dn, preferred_element_type=jnp.float32)
    msgs = t1 * inv + gate * (t2 + bm_ref[...])
    out = lax.dot_general(msgs, wu_ref[...], dn, preferred_element_type=jnp.float32)
    o_ref[...] = out + bu_ref[...]


def _tc_dense(xb, acc, W_msg, b_msg, W_upd, b_upd):
    blk = 1000
    grid = N_NODES // blk
    return pl.pallas_call(
        _tc_dense_body,
        grid=(grid,),
        in_specs=[
            pl.BlockSpec((blk, D_IN), lambda i: (i, 0)),
            pl.BlockSpec((2, blk, D_AUG), lambda i: (0, i, 0)),
            pl.BlockSpec((D_IN, 2 * D_IN), lambda i: (0, 0)),
            pl.BlockSpec((1, D_IN), lambda i: (0, 0)),
            pl.BlockSpec((D_IN, D_IN), lambda i: (0, 0)),
            pl.BlockSpec((1, D_IN), lambda i: (0, 0)),
        ],
        out_specs=pl.BlockSpec((blk, D_IN), lambda i: (i, 0)),
        out_shape=jax.ShapeDtypeStruct((N_NODES, D_IN), jnp.float32),
    )(xb, acc, W_msg, b_msg, W_upd, b_upd)


@jax.jit
def kernel(x, edge_index, W_msg, b_msg, W_upd, b_upd):
    xb = x[0]
    src = edge_index[0].astype(jnp.int32)
    dst = edge_index[1].astype(jnp.int32)
    npad_e = E_PAD - N_EDGES
    src3 = jnp.concatenate(
        [src, jnp.full((npad_e,), N_TAB - 1, jnp.int32)]).reshape(
        NUM_WORKERS, NUM_CHUNKS, CHUNK)
    dst3 = jnp.concatenate(
        [dst, jnp.arange(npad_e, dtype=jnp.int32) % N_NODES]).reshape(
        NUM_WORKERS, NUM_CHUNKS, CHUNK)

    xa = jnp.zeros((N_TAB, D_AUG), jnp.float32)
    xa = xa.at[:N_NODES, :D_IN].set(xb)
    xa = xa.at[:N_NODES, D_IN].set(1.0)
    zblk = jnp.zeros((ROWS_PER_SUBCORE, D_AUG), jnp.float32)

    acc = _sc_segment_sum(xa, src3, dst3, zblk)

    out = _tc_dense(xb, acc, W_msg,
                    b_msg.reshape(1, D_IN), W_upd, b_upd.reshape(1, D_IN))
    return out[None]


# speedup vs baseline: 1.9549x; 1.2718x over previous
import functools

import jax
import jax.numpy as jnp
from jax import lax
from jax.experimental import pallas as pl
from jax.experimental.pallas import tpu as pltpu
from jax.experimental.pallas import tpu_sc as plsc

N_NODES = 10000
N_TAB = 10240
D_IN = 128
D_AUG = 144
N_EDGES = 320000
NUM_WORKERS = 32
CHUNK = 80
NUM_CHUNKS = 126
EDGES_PER_WORKER = NUM_CHUNKS * CHUNK
E_PAD = NUM_WORKERS * EDGES_PER_WORKER
ROWS_PER_SUBCORE = N_NODES // 16


def _sc_segment_sum(xa, src3, dst3, zblk):
    mesh = plsc.VectorSubcoreMesh(core_axis_name="c", subcore_axis_name="s")

    @functools.partial(
        pl.kernel,
        out_type=jax.ShapeDtypeStruct((2, N_NODES, D_AUG), jnp.float32),
        mesh=mesh,
        scratch_types=[
            pltpu.VMEM((NUM_CHUNKS, CHUNK), jnp.int32),
            pltpu.VMEM((NUM_CHUNKS, CHUNK), jnp.int32),
            pltpu.VMEM((CHUNK, D_AUG), jnp.float32),
            pltpu.VMEM_SHARED((N_NODES, D_AUG), jnp.float32),
            pltpu.SemaphoreType.DMA,
        ],
        compiler_params=pltpu.CompilerParams(use_tc_tiling_on_sc=False),
    )
    def seg_sum(xa_hbm, src_hbm, dst_hbm, zblk_hbm, out_hbm,
                src_v, dst_v, rows_v, acc_sh, gsem):
        c = lax.axis_index("c")
        s = lax.axis_index("s")
        wid = s * 2 + c
        row0 = s * ROWS_PER_SUBCORE

        pltpu.sync_copy(zblk_hbm, acc_sh.at[pl.ds(row0, ROWS_PER_SUBCORE), :])
        pltpu.sync_copy(src_hbm.at[wid], src_v)
        pltpu.sync_copy(dst_hbm.at[wid], dst_v)
        plsc.subcore_barrier()

        def chunk_body(g, carry):
            pltpu.async_copy(xa_hbm.at[src_v.at[g]], rows_v, gsem).wait()
            pltpu.sync_copy(rows_v, acc_sh.at[dst_v.at[g]], add=True)
            return carry

        lax.fori_loop(0, NUM_CHUNKS, chunk_body, 0)
        plsc.subcore_barrier()

        pltpu.sync_copy(acc_sh.at[pl.ds(row0, ROWS_PER_SUBCORE), :],
                        out_hbm.at[c, pl.ds(row0, ROWS_PER_SUBCORE), :])

    return seg_sum(xa, src3, dst3, zblk)


def _tc_dense_body(x_ref, a_ref, wm_ref, bm_ref, wu_ref, bu_ref, o_ref):
    asum = a_ref[0] + a_ref[1]
    feat = asum[:, :D_IN]
    cnt = asum[:, D_IN:D_IN + 1]
    inv = 1.0 / jnp.maximum(cnt, 1.0)
    gate = cnt * inv
    w1 = wm_ref[:, :D_IN]
    w2 = wm_ref[:, D_IN:]
    dn = (((1,), (1,)), ((), ()))
    t1 = lax.dot_general(feat, w1, dn, preferred_element_type=jnp.float32)
    t2 = lax.dot_general(x_ref[...], w2, dn, preferred_element_type=jnp.float32)
    msgs = t1 * inv + gate * (t2 + bm_ref[...])
    out = lax.dot_general(msgs, wu_ref[...], dn, preferred_element_type=jnp.float32)
    o_ref[...] = out + bu_ref[...]


def _tc_dense(xb, acc, W_msg, b_msg, W_upd, b_upd):
    blk = 1000
    grid = N_NODES // blk
    return pl.pallas_call(
        _tc_dense_body,
        grid=(grid,),
        in_specs=[
            pl.BlockSpec((blk, D_IN), lambda i: (i, 0)),
            pl.BlockSpec((2, blk, D_AUG), lambda i: (0, i, 0)),
            pl.BlockSpec((D_IN, 2 * D_IN), lambda i: (0, 0)),
            pl.BlockSpec((1, D_IN), lambda i: (0, 0)),
            pl.BlockSpec((D_IN, D_IN), lambda i: (0, 0)),
            pl.BlockSpec((1, D_IN), lambda i: (0, 0)),
        ],
        out_specs=pl.BlockSpec((blk, D_IN), lambda i: (i, 0)),
        out_shape=jax.ShapeDtypeStruct((N_NODES, D_IN), jnp.float32),
    )(xb, acc, W_msg, b_msg, W_upd, b_upd)


@jax.jit
def kernel(x, edge_index, W_msg, b_msg, W_upd, b_upd):
    xb = x[0]
    src = edge_index[0].astype(jnp.int32)
    dst = edge_index[1].astype(jnp.int32)
    npad_e = E_PAD - N_EDGES
    src3 = jnp.concatenate(
        [src, N_NODES + jnp.arange(npad_e, dtype=jnp.int32) %
         (N_TAB - N_NODES)]).reshape(NUM_WORKERS, NUM_CHUNKS, CHUNK)
    dst3 = jnp.concatenate(
        [dst, jnp.arange(npad_e, dtype=jnp.int32) % N_NODES]).reshape(
        NUM_WORKERS, NUM_CHUNKS, CHUNK)

    xa = jnp.zeros((N_TAB, D_AUG), jnp.float32)
    xa = xa.at[:N_NODES, :D_IN].set(xb)
    xa = xa.at[:N_NODES, D_IN].set(1.0)
    zblk = jnp.zeros((ROWS_PER_SUBCORE, D_AUG), jnp.float32)

    acc = _sc_segment_sum(xa, src3, dst3, zblk)

    out = _tc_dense(xb, acc, W_msg,
                    b_msg.reshape(1, D_IN), W_upd, b_upd.reshape(1, D_IN))
    return out[None]


# speedup vs baseline: 2.6697x; 1.3657x over previous
import functools

import jax
import jax.numpy as jnp
from jax import lax
from jax.experimental import pallas as pl
from jax.experimental.pallas import tpu as pltpu
from jax.experimental.pallas import tpu_sc as plsc

N_NODES = 10000
N_TAB = 10240
D_IN = 128
D_AUG = 144
N_EDGES = 320000
NUM_WORKERS = 32
CHUNK = 72
NUM_CHUNKS = 140
EDGES_PER_WORKER = NUM_CHUNKS * CHUNK
E_PAD = NUM_WORKERS * EDGES_PER_WORKER
ROWS_PER_SUBCORE = N_NODES // 16


def _sc_segment_sum(xa, src3, dst3, zblk):
    mesh = plsc.VectorSubcoreMesh(core_axis_name="c", subcore_axis_name="s")

    @functools.partial(
        pl.kernel,
        out_type=jax.ShapeDtypeStruct((2, N_NODES, D_AUG), jnp.float32),
        mesh=mesh,
        scratch_types=[
            pltpu.VMEM((NUM_CHUNKS, CHUNK), jnp.int32),
            pltpu.VMEM((NUM_CHUNKS, CHUNK), jnp.int32),
            pltpu.VMEM((2, CHUNK, D_AUG), jnp.float32),
            pltpu.VMEM_SHARED((N_NODES, D_AUG), jnp.float32),
            [pltpu.SemaphoreType.DMA] * 2,
        ],
        compiler_params=pltpu.CompilerParams(use_tc_tiling_on_sc=False),
    )
    def seg_sum(xa_hbm, src_hbm, dst_hbm, zblk_hbm, out_hbm,
                src_v, dst_v, rows_v, acc_sh, gsems):
        c = lax.axis_index("c")
        s = lax.axis_index("s")
        wid = s * 2 + c
        row0 = s * ROWS_PER_SUBCORE

        pltpu.sync_copy(zblk_hbm, acc_sh.at[pl.ds(row0, ROWS_PER_SUBCORE), :])
        pltpu.sync_copy(src_hbm.at[wid], src_v)
        pltpu.sync_copy(dst_hbm.at[wid], dst_v)
        plsc.subcore_barrier()

        desc = [None] * NUM_CHUNKS
        desc[0] = pltpu.async_copy(xa_hbm.at[src_v.at[0]], rows_v.at[0],
                                   gsems[0])
        for g in range(NUM_CHUNKS):
            if g + 1 < NUM_CHUNKS:
                desc[g + 1] = pltpu.async_copy(
                    xa_hbm.at[src_v.at[g + 1]], rows_v.at[(g + 1) % 2],
                    gsems[(g + 1) % 2])
            desc[g].wait()
            pltpu.sync_copy(rows_v.at[g % 2], acc_sh.at[dst_v.at[g]],
                            add=True)
        plsc.subcore_barrier()

        pltpu.sync_copy(acc_sh.at[pl.ds(row0, ROWS_PER_SUBCORE), :],
                        out_hbm.at[c, pl.ds(row0, ROWS_PER_SUBCORE), :])

    return seg_sum(xa, src3, dst3, zblk)


def _tc_dense_body(x_ref, a_ref, wm_ref, bm_ref, wu_ref, bu_ref, o_ref):
    asum = a_ref[0] + a_ref[1]
    feat = asum[:, :D_IN]
    cnt = asum[:, D_IN:D_IN + 1]
    inv = 1.0 / jnp.maximum(cnt, 1.0)
    gate = cnt * inv
    w1 = wm_ref[:, :D_IN]
    w2 = wm_ref[:, D_IN:]
    dn = (((1,), (1,)), ((), ()))
    t1 = lax.dot_general(feat, w1, dn, preferred_element_type=jnp.float32)
    t2 = lax.dot_general(x_ref[...], w2, dn, preferred_element_type=jnp.float32)
    msgs = t1 * inv + gate * (t2 + bm_ref[...])
    out = lax.dot_general(msgs, wu_ref[...], dn, preferred_element_type=jnp.float32)
    o_ref[...] = out + bu_ref[...]


def _tc_dense(xb, acc, W_msg, b_msg, W_upd, b_upd):
    blk = 1000
    grid = N_NODES // blk
    return pl.pallas_call(
        _tc_dense_body,
        grid=(grid,),
        in_specs=[
            pl.BlockSpec((blk, D_IN), lambda i: (i, 0)),
            pl.BlockSpec((2, blk, D_AUG), lambda i: (0, i, 0)),
            pl.BlockSpec((D_IN, 2 * D_IN), lambda i: (0, 0)),
            pl.BlockSpec((1, D_IN), lambda i: (0, 0)),
            pl.BlockSpec((D_IN, D_IN), lambda i: (0, 0)),
            pl.BlockSpec((1, D_IN), lambda i: (0, 0)),
        ],
        out_specs=pl.BlockSpec((blk, D_IN), lambda i: (i, 0)),
        out_shape=jax.ShapeDtypeStruct((N_NODES, D_IN), jnp.float32),
    )(xb, acc, W_msg, b_msg, W_upd, b_upd)


@jax.jit
def kernel(x, edge_index, W_msg, b_msg, W_upd, b_upd):
    xb = x[0]
    src = edge_index[0].astype(jnp.int32)
    dst = edge_index[1].astype(jnp.int32)
    npad_e = E_PAD - N_EDGES
    src3 = jnp.concatenate(
        [src, N_NODES + jnp.arange(npad_e, dtype=jnp.int32) %
         (N_TAB - N_NODES)]).reshape(NUM_WORKERS, NUM_CHUNKS, CHUNK)
    dst3 = jnp.concatenate(
        [dst, jnp.arange(npad_e, dtype=jnp.int32) % N_NODES]).reshape(
        NUM_WORKERS, NUM_CHUNKS, CHUNK)

    xa = jnp.zeros((N_TAB, D_AUG), jnp.float32)
    xa = xa.at[:N_NODES, :D_IN].set(xb)
    xa = xa.at[:N_NODES, D_IN].set(1.0)
    zblk = jnp.zeros((ROWS_PER_SUBCORE, D_AUG), jnp.float32)

    acc = _sc_segment_sum(xa, src3, dst3, zblk)

    out = _tc_dense(xb, acc, W_msg,
                    b_msg.reshape(1, D_IN), W_upd, b_upd.reshape(1, D_IN))
    return out[None]


# speedup vs baseline: 2.6710x; 1.0005x over previous
import functools

import jax
import jax.numpy as jnp
from jax import lax
from jax.experimental import pallas as pl
from jax.experimental.pallas import tpu as pltpu
from jax.experimental.pallas import tpu_sc as plsc

N_NODES = 10000
N_TAB = 10240
D_IN = 128
D_AUG = 144
N_EDGES = 320000
NUM_WORKERS = 32
CHUNK = 72
NUM_CHUNKS = 140
EDGES_PER_WORKER = NUM_CHUNKS * CHUNK
E_PAD = NUM_WORKERS * EDGES_PER_WORKER
ROWS_PER_SUBCORE = N_NODES // 16


def _sc_segment_sum(xa, src3, dst3, zblk):
    mesh = plsc.VectorSubcoreMesh(core_axis_name="c", subcore_axis_name="s")

    @functools.partial(
        pl.kernel,
        out_type=jax.ShapeDtypeStruct((2, N_NODES, D_AUG), jnp.float32),
        mesh=mesh,
        scratch_types=[
            pltpu.VMEM((NUM_CHUNKS, CHUNK), jnp.int32),
            pltpu.VMEM((NUM_CHUNKS, CHUNK), jnp.int32),
            pltpu.VMEM((2, CHUNK, D_AUG), jnp.float32),
            pltpu.VMEM_SHARED((N_NODES, D_AUG), jnp.float32),
            [pltpu.SemaphoreType.DMA] * 2,
            [pltpu.SemaphoreType.DMA] * 2,
        ],
        compiler_params=pltpu.CompilerParams(use_tc_tiling_on_sc=False),
    )
    def seg_sum(xa_hbm, src_hbm, dst_hbm, zblk_hbm, out_hbm,
                src_v, dst_v, rows_v, acc_sh, gsems, ssems):
        c = lax.axis_index("c")
        s = lax.axis_index("s")
        wid = s * 2 + c
        row0 = s * ROWS_PER_SUBCORE

        pltpu.sync_copy(zblk_hbm, acc_sh.at[pl.ds(row0, ROWS_PER_SUBCORE), :])
        pltpu.sync_copy(src_hbm.at[wid], src_v)
        pltpu.sync_copy(dst_hbm.at[wid], dst_v)
        plsc.subcore_barrier()

        gdesc = [None] * NUM_CHUNKS
        sdesc = [None] * NUM_CHUNKS
        gdesc[0] = pltpu.async_copy(xa_hbm.at[src_v.at[0]], rows_v.at[0],
                                    gsems[0])
        for g in range(NUM_CHUNKS):
            if g + 1 < NUM_CHUNKS:
                if g >= 1:
                    sdesc[g - 1].wait()
                gdesc[g + 1] = pltpu.async_copy(
                    xa_hbm.at[src_v.at[g + 1]], rows_v.at[(g + 1) % 2],
                    gsems[(g + 1) % 2])
            gdesc[g].wait()
            sdesc[g] = pltpu.async_copy(rows_v.at[g % 2],
                                        acc_sh.at[dst_v.at[g]],
                                        ssems[g % 2], add=True)
        sdesc[NUM_CHUNKS - 2].wait()
        sdesc[NUM_CHUNKS - 1].wait()
        plsc.subcore_barrier()

        pltpu.sync_copy(acc_sh.at[pl.ds(row0, ROWS_PER_SUBCORE), :],
                        out_hbm.at[c, pl.ds(row0, ROWS_PER_SUBCORE), :])

    return seg_sum(xa, src3, dst3, zblk)


def _tc_dense_body(x_ref, a_ref, wm_ref, bm_ref, wu_ref, bu_ref, o_ref):
    asum = a_ref[0] + a_ref[1]
    feat = asum[:, :D_IN]
    cnt = asum[:, D_IN:D_IN + 1]
    inv = 1.0 / jnp.maximum(cnt, 1.0)
    gate = cnt * inv
    w1 = wm_ref[:, :D_IN]
    w2 = wm_ref[:, D_IN:]
    dn = (((1,), (1,)), ((), ()))
    t1 = lax.dot_general(feat, w1, dn, preferred_element_type=jnp.float32)
    t2 = lax.dot_general(x_ref[...], w2, dn, preferred_element_type=jnp.float32)
    msgs = t1 * inv + gate * (t2 + bm_ref[...])
    out = lax.dot_general(msgs, wu_ref[...], dn, preferred_element_type=jnp.float32)
    o_ref[...] = out + bu_ref[...]


def _tc_dense(xb, acc, W_msg, b_msg, W_upd, b_upd):
    blk = 1000
    grid = N_NODES // blk
    return pl.pallas_call(
        _tc_dense_body,
        grid=(grid,),
        in_specs=[
            pl.BlockSpec((blk, D_IN), lambda i: (i, 0)),
            pl.BlockSpec((2, blk, D_AUG), lambda i: (0, i, 0)),
            pl.BlockSpec((D_IN, 2 * D_IN), lambda i: (0, 0)),
            pl.BlockSpec((1, D_IN), lambda i: (0, 0)),
            pl.BlockSpec((D_IN, D_IN), lambda i: (0, 0)),
            pl.BlockSpec((1, D_IN), lambda i: (0, 0)),
        ],
        out_specs=pl.BlockSpec((blk, D_IN), lambda i: (i, 0)),
        out_shape=jax.ShapeDtypeStruct((N_NODES, D_IN), jnp.float32),
    )(xb, acc, W_msg, b_msg, W_upd, b_upd)


@jax.jit
def kernel(x, edge_index, W_msg, b_msg, W_upd, b_upd):
    xb = x[0]
    src = edge_index[0].astype(jnp.int32)
    dst = edge_index[1].astype(jnp.int32)
    npad_e = E_PAD - N_EDGES
    src3 = jnp.concatenate(
        [src, N_NODES + jnp.arange(npad_e, dtype=jnp.int32) %
         (N_TAB - N_NODES)]).reshape(NUM_WORKERS, NUM_CHUNKS, CHUNK)
    dst3 = jnp.concatenate(
        [dst, jnp.arange(npad_e, dtype=jnp.int32) % N_NODES]).reshape(
        NUM_WORKERS, NUM_CHUNKS, CHUNK)

    xa = jnp.zeros((N_TAB, D_AUG), jnp.float32)
    xa = xa.at[:N_NODES, :D_IN].set(xb)
    xa = xa.at[:N_NODES, D_IN].set(1.0)
    zblk = jnp.zeros((ROWS_PER_SUBCORE, D_AUG), jnp.float32)

    acc = _sc_segment_sum(xa, src3, dst3, zblk)

    out = _tc_dense(xb, acc, W_msg,
                    b_msg.reshape(1, D_IN), W_upd, b_upd.reshape(1, D_IN))
    return out[None]
